# Initial kernel scaffold; baseline (speedup 1.0000x reference)
import jax
import jax.numpy as jnp
from jax.experimental import pallas as pl


def kernel(logits_exp, logits_ama):
    raise NotImplementedError("write your pallas kernel here")


# speedup vs baseline: 23.0779x; 23.0779x over previous
import functools
from math import ceil

import jax
import jax.numpy as jnp
from jax.experimental import pallas as pl

ALPHA = 0.9
K_SEL = 40

def _to_ordered_int(x):
    i = jax.lax.bitcast_convert_type(x, jnp.int32)
    return i ^ ((i >> 31) & jnp.int32(0x7FFFFFFF))


def _from_ordered_int(s):
    i = s ^ ((s >> 31) & jnp.int32(0x7FFFFFFF))
    return jax.lax.bitcast_convert_type(i, jnp.float32)


def _count_ge(s, thr):
    return jnp.sum((s >= thr).astype(jnp.int32), axis=1, keepdims=True)


def _kth_thresholds(s, ks):
    R = s.shape[0]
    thrs = []
    for k in ks:
        cnt = _count_ge(s, jnp.zeros((R, 1), jnp.int32))
        thrs.append(jnp.where(cnt >= k, jnp.int32(0), jnp.int32(-0x80000000)))

    def body(it, carry):
        b = 30 - it
        bit = (jnp.int32(1) << b)
        new = []
        for T, k in zip(carry, ks):
            cand = T | bit
            cnt = _count_ge(s, cand)
            new.append(jnp.where(cnt >= k, cand, T))
        return tuple(new)

    thrs = jax.lax.fori_loop(0, 31, body, tuple(thrs))
    return list(thrs)


def _tie_index_cutoff(tie, idx, need, bits):
    R = tie.shape[0]
    C0 = jnp.zeros((R, 1), jnp.int32)

    def body(it, C):
        b = bits - 1 - it
        cand = C | ((jnp.int32(1) << b) - 1)
        cnt = jnp.sum((tie & (idx <= cand)).astype(jnp.int32), axis=1,
                      keepdims=True)
        return jnp.where(cnt >= need, C, C | (jnp.int32(1) << b))

    return jax.lax.fori_loop(0, bits, body, C0)


def _ctk_kernel(exp_ref, ama_ref, out_ref, *, k_big, k_sel):
    le = exp_ref[...]
    la = ama_ref[...]
    me = jnp.max(le, axis=1, keepdims=True)
    ma = jnp.max(la, axis=1, keepdims=True)

    se = _to_ordered_int(le)
    sa = _to_ordered_int(la)

    t_e, t40 = _kth_thresholds(se, (k_big, k_sel))
    (t_a,) = _kth_thresholds(sa, (k_big,))

    ee = jnp.exp(le - me)
    ea = jnp.exp(la - ma)

    cge_e = _count_ge(se, t_e).astype(jnp.float32)
    cge_a = _count_ge(sa, t_a).astype(jnp.float32)
    tf_e = _from_ordered_int(t_e)
    tf_a = _from_ordered_int(t_a)
    z_e = (jnp.sum(jnp.where(se >= t_e, ee, 0.0), axis=1, keepdims=True)
           - (cge_e - k_big) * jnp.exp(tf_e - me))
    z_a = (jnp.sum(jnp.where(sa >= t_a, ea, 0.0), axis=1, keepdims=True)
           - (cge_a - k_big) * jnp.exp(tf_a - ma))

    p_exp = ee / z_e
    p_ama = jnp.where(sa >= t_a, ea, 0.0) / z_a
    score = jnp.log(p_exp / (p_ama + jnp.float32(1e-8)))

    v = se.shape[1]
    bits = max(1, (v - 1).bit_length())
    idx = jax.lax.broadcasted_iota(jnp.int32, se.shape, 1)
    need = k_sel - jnp.sum((se > t40).astype(jnp.int32), axis=1, keepdims=True)
    cutoff = _tie_index_cutoff(se == t40, idx, need, bits)
    sel = (se > t40) | ((se == t40) & (idx <= cutoff))
    out_ref[...] = jnp.where(sel, score, float("-inf"))


@jax.jit
def kernel(logits_exp, logits_ama):
    n, v = logits_exp.shape
    k_big = int(ceil((1.0 - ALPHA) * v))
    rows_per_block = 8
    grid = (n // rows_per_block,)
    spec = pl.BlockSpec((rows_per_block, v), lambda i: (i, 0))
    return pl.pallas_call(
        functools.partial(_ctk_kernel, k_big=k_big, k_sel=K_SEL),
        grid=grid,
        in_specs=[spec, spec],
        out_specs=spec,
        out_shape=jax.ShapeDtypeStruct((n, v), jnp.float32),
    )(logits_exp, logits_ama)
